# baseline (speedup 1.0000x reference)
import jax
import jax.numpy as jnp
from jax import lax
from jax.experimental import pallas as pl
from jax.experimental.pallas import tpu as pltpu
from jax.experimental.pallas import tpu_sc as plsc

VOCAB = 100000
DIM = 768
SEQ = 2048
BATCH = 4

NC = 2
NS = 16
LANES = 16
NW = NC * NS
POSB = SEQ // NW
CHUNK = 8
NGRP = POSB // CHUNK
DSLICES = DIM // LANES


def _body(ids_hbm, vocab_hbm, pos_hbm, out_hbm, idx_v, pos_v,
          r00, r01, r10, r11, r20, r21, r30, r31,
          psem, isem,
          gs00, gs01, gs10, gs11, gs20, gs21, gs30, gs31,
          os00, os01, os10, os11, os20, os21, os30, os31):
    rows = [[r00, r01], [r10, r11], [r20, r21], [r30, r31]]
    gsem = [[gs00, gs01], [gs10, gs11], [gs20, gs21], [gs30, gs31]]
    osem = [[os00, os01], [os10, os11], [os20, os21], [os30, os31]]

    c_i = lax.axis_index("c")
    s_i = lax.axis_index("s")
    wid = s_i * NC + c_i
    pos_base = wid * POSB

    pload = pltpu.async_copy(pos_hbm.at[pl.ds(pos_base, POSB)], pos_v, psem)
    iloads = [pltpu.async_copy(ids_hbm.at[b, pl.ds(pos_base, POSB)],
                               idx_v.at[b], isem) for b in range(BATCH)]
    for il in iloads:
        il.wait()

    g, o = {}, {}

    def gathers(h):
        par = h % 2
        for b in range(BATCH):
            g[(b, h)] = pltpu.async_copy(
                vocab_hbm.at[idx_v.at[b, pl.ds(h * CHUNK, CHUNK)]],
                rows[b][par], gsem[b][par])

    def outs(h):
        par = h % 2
        for b in range(BATCH):
            o[(b, h)] = pltpu.async_copy(
                rows[b][par],
                out_hbm.at[pl.ds(b * SEQ + pos_base + h * CHUNK, CHUNK)],
                osem[b][par])

    gathers(0)
    pload.wait()
    for h in range(NGRP):
        par = h % 2
        if h + 1 < NGRP:
            if h >= 1:
                for b in range(BATCH):
                    o[(b, h - 1)].wait()
            gathers(h + 1)
        for b in range(BATCH):
            g[(b, h)].wait()

        def add_group(par=par, h=h):
            @plsc.parallel_loop(0, CHUNK)
            def _row(r):
                rbufs = [rows[b][par].at[r] for b in range(BATCH)]
                pr = pos_v.at[h * CHUNK + r]
                for d in range(DSLICES):
                    sl = pl.ds(d * LANES, LANES)
                    pv = pr[sl]
                    for b in range(BATCH):
                        plsc.addupdate(rbufs[b].at[sl], pv)

        add_group()
        outs(h)
    for b in range(BATCH):
        o[(b, NGRP - 2)].wait()
        o[(b, NGRP - 1)].wait()


@jax.jit
def kernel(input_ids, vocab_W, pos_W):
    ids = input_ids.astype(jnp.int32)
    mesh = plsc.VectorSubcoreMesh(core_axis_name="c", subcore_axis_name="s")
    run = pl.kernel(
        _body,
        out_type=jax.ShapeDtypeStruct((BATCH * SEQ, DIM), jnp.float32),
        mesh=mesh,
        scratch_types=(
            [pltpu.VMEM((BATCH, POSB), jnp.int32),
             pltpu.VMEM((POSB, DIM), jnp.float32)]
            + [pltpu.VMEM((CHUNK, DIM), jnp.float32)
               for _ in range(2 * BATCH)]
            + [pltpu.SemaphoreType.DMA for _ in range(2 + 4 * BATCH)]
        ),
    )
    out = run(ids, vocab_W, pos_W)
    return out.reshape(BATCH, SEQ, DIM)


# speedup vs baseline: 1.2136x; 1.2136x over previous
import jax
import jax.numpy as jnp
from jax import lax
from jax.experimental import pallas as pl
from jax.experimental.pallas import tpu as pltpu
from jax.experimental.pallas import tpu_sc as plsc

VOCAB = 100000
DIM = 768
SEQ = 2048
BATCH = 4

NC = 2
NS = 16
LANES = 16
NW = NC * NS
POSB = SEQ // NW
CHUNK = 8
NGRP = POSB // CHUNK
DSLICES = DIM // LANES


def _body(ids_hbm, vocab_hbm, pos_hbm, out_hbm, idx_v, pos_v,
          r00, r01, r10, r11, r20, r21, r30, r31,
          psem, isem,
          gs00, gs01, gs10, gs11, gs20, gs21, gs30, gs31,
          os00, os01, os10, os11, os20, os21, os30, os31):
    rows = [[r00, r01], [r10, r11], [r20, r21], [r30, r31]]
    gsem = [[gs00, gs01], [gs10, gs11], [gs20, gs21], [gs30, gs31]]
    osem = [[os00, os01], [os10, os11], [os20, os21], [os30, os31]]

    c_i = lax.axis_index("c")
    s_i = lax.axis_index("s")
    wid = s_i * NC + c_i
    pos_base = wid * POSB

    pload = pltpu.async_copy(pos_hbm.at[pl.ds(pos_base, POSB)], pos_v, psem)
    iloads = [pltpu.async_copy(ids_hbm.at[b, pl.ds(pos_base, POSB)],
                               idx_v.at[b], isem) for b in range(BATCH)]
    for il in iloads:
        il.wait()

    g, o = {}, {}

    def gathers(h):
        par = h % 2
        for b in range(BATCH):
            g[(b, h)] = pltpu.async_copy(
                vocab_hbm.at[idx_v.at[b, pl.ds(h * CHUNK, CHUNK)]],
                rows[b][par], gsem[b][par])

    def outs(h):
        par = h % 2
        for b in range(BATCH):
            o[(b, h)] = pltpu.async_copy(
                rows[b][par],
                out_hbm.at[pl.ds(b * SEQ + pos_base + h * CHUNK, CHUNK)],
                osem[b][par])

    gathers(0)
    pload.wait()
    for h in range(NGRP):
        par = h % 2
        if h + 1 < NGRP:
            if h >= 1:
                for b in range(BATCH):
                    o[(b, h - 1)].wait()
            gathers(h + 1)
        for b in range(BATCH):
            g[(b, h)].wait()

        def add_group(par=par, h=h):
            @plsc.parallel_loop(0, CHUNK)
            def _row(r):
                rbufs = [rows[b][par].at[r] for b in range(BATCH)]
                pr = pos_v.at[h * CHUNK + r]
                @plsc.parallel_loop(0, DIM, step=LANES, unroll=4)
                def _slice(dd):
                    sl = pl.ds(dd, LANES)
                    pv = pr[sl]
                    for b in range(BATCH):
                        rbufs[b][sl] = rbufs[b][sl] + pv

        add_group()
        outs(h)
    for b in range(BATCH):
        o[(b, NGRP - 2)].wait()
        o[(b, NGRP - 1)].wait()


@jax.jit
def kernel(input_ids, vocab_W, pos_W):
    ids = input_ids.astype(jnp.int32)
    mesh = plsc.VectorSubcoreMesh(core_axis_name="c", subcore_axis_name="s")
    run = pl.kernel(
        _body,
        out_type=jax.ShapeDtypeStruct((BATCH * SEQ, DIM), jnp.float32),
        mesh=mesh,
        scratch_types=(
            [pltpu.VMEM((BATCH, POSB), jnp.int32),
             pltpu.VMEM((POSB, DIM), jnp.float32)]
            + [pltpu.VMEM((CHUNK, DIM), jnp.float32)
               for _ in range(2 * BATCH)]
            + [pltpu.SemaphoreType.DMA for _ in range(2 + 4 * BATCH)]
        ),
    )
    out = run(ids, vocab_W, pos_W)
    return out.reshape(BATCH, SEQ, DIM)


# speedup vs baseline: 1.2302x; 1.0137x over previous
import jax
import jax.numpy as jnp
from jax import lax
from jax.experimental import pallas as pl
from jax.experimental.pallas import tpu as pltpu
from jax.experimental.pallas import tpu_sc as plsc

VOCAB = 100000
DIM = 768
SEQ = 2048
BATCH = 4

NC = 2
NS = 16
LANES = 16
NW = NC * NS
POSB = SEQ // NW
CHUNK = 16
HCHUNK = POSB // CHUNK
NCHUNK = BATCH * HCHUNK
DSLICES = DIM // LANES
NRB = 4


def _body(ids_hbm, vocab_hbm, pos_hbm, out_hbm, idx_v, pos_v,
          r0, r1, r2, r3, psem, isem, gs0, gs1, gs2, gs3, os0, os1, os2, os3):
    rows = [r0, r1, r2, r3]
    gsem = [gs0, gs1, gs2, gs3]
    osem = [os0, os1, os2, os3]

    c_i = lax.axis_index("c")
    s_i = lax.axis_index("s")
    wid = s_i * NC + c_i
    pos_base = wid * POSB

    pload = pltpu.async_copy(pos_hbm.at[pl.ds(pos_base, POSB)], pos_v, psem)
    iloads = [pltpu.async_copy(ids_hbm.at[b, pl.ds(pos_base, POSB)],
                               idx_v.at[b], isem) for b in range(BATCH)]
    for il in iloads:
        il.wait()

    g, o = {}, {}

    def start(c):
        b, h = c // HCHUNK, c % HCHUNK
        rb = c % NRB
        g[c] = pltpu.async_copy(
            vocab_hbm.at[idx_v.at[b, pl.ds(h * CHUNK, CHUNK)]],
            rows[rb], gsem[rb])

    start(0)
    start(1)
    pload.wait()
    for c in range(NCHUNK):
        if c + 2 < NCHUNK:
            if c >= 2:
                o[c - 2].wait()
            start(c + 2)
        g[c].wait()
        b, h = c // HCHUNK, c % HCHUNK
        rb = c % NRB

        def add_rows(rb=rb, h=h):
            @plsc.parallel_loop(0, CHUNK)
            def _row(r):
                rr = rows[rb].at[r]
                pr = pos_v.at[h * CHUNK + r]

                @plsc.parallel_loop(0, DIM, step=LANES, unroll=8)
                def _slice(dd):
                    sl = pl.ds(dd, LANES)
                    plsc.addupdate(rr.at[sl], pr[sl])

        add_rows()
        o[c] = pltpu.async_copy(
            rows[rb],
            out_hbm.at[pl.ds(b * SEQ + pos_base + h * CHUNK, CHUNK)],
            osem[rb])
    for c in range(NCHUNK - 4, NCHUNK):
        o[c].wait()


@jax.jit
def kernel(input_ids, vocab_W, pos_W):
    ids = input_ids.astype(jnp.int32)
    mesh = plsc.VectorSubcoreMesh(core_axis_name="c", subcore_axis_name="s")
    run = pl.kernel(
        _body,
        out_type=jax.ShapeDtypeStruct((BATCH * SEQ, DIM), jnp.float32),
        mesh=mesh,
        scratch_types=(
            [pltpu.VMEM((BATCH, POSB), jnp.int32),
             pltpu.VMEM((POSB, DIM), jnp.float32)]
            + [pltpu.VMEM((CHUNK, DIM), jnp.float32) for _ in range(NRB)]
            + [pltpu.SemaphoreType.DMA for _ in range(2 + 2 * NRB)]
        ),
    )
    out = run(ids, vocab_W, pos_W)
    return out.reshape(BATCH, SEQ, DIM)


# speedup vs baseline: 1.2391x; 1.0073x over previous
import jax
import jax.numpy as jnp
from jax import lax
from jax.experimental import pallas as pl
from jax.experimental.pallas import tpu as pltpu
from jax.experimental.pallas import tpu_sc as plsc

VOCAB = 100000
DIM = 768
SEQ = 2048
BATCH = 4

NC = 2
NS = 16
LANES = 16
NW = NC * NS
POSB = SEQ // NW
CHUNK = 8
NGRP = POSB // CHUNK
DSLICES = DIM // LANES


def _body(ids_hbm, vocab_hbm, pos_hbm, out_hbm, idx_v, pos_v,
          r00, r01, r02, r10, r11, r12, r20, r21, r22, r30, r31, r32,
          psem, isem,
          gs00, gs01, gs02, gs10, gs11, gs12, gs20, gs21, gs22,
          gs30, gs31, gs32,
          os00, os01, os02, os10, os11, os12, os20, os21, os22,
          os30, os31, os32):
    rows = [[r00, r01, r02], [r10, r11, r12], [r20, r21, r22],
            [r30, r31, r32]]
    gsem = [[gs00, gs01, gs02], [gs10, gs11, gs12], [gs20, gs21, gs22],
            [gs30, gs31, gs32]]
    osem = [[os00, os01, os02], [os10, os11, os12], [os20, os21, os22],
            [os30, os31, os32]]

    c_i = lax.axis_index("c")
    s_i = lax.axis_index("s")
    wid = s_i * NC + c_i
    pos_base = wid * POSB

    pload = pltpu.async_copy(pos_hbm.at[pl.ds(pos_base, POSB)], pos_v, psem)
    iloads = [pltpu.async_copy(ids_hbm.at[b, pl.ds(pos_base, POSB)],
                               idx_v.at[b], isem) for b in range(BATCH)]
    for il in iloads:
        il.wait()

    g, o = {}, {}

    def gathers(h):
        par = h % 3
        for b in range(BATCH):
            g[(b, h)] = pltpu.async_copy(
                vocab_hbm.at[idx_v.at[b, pl.ds(h * CHUNK, CHUNK)]],
                rows[b][par], gsem[b][par])

    def outs(h):
        par = h % 3
        for b in range(BATCH):
            o[(b, h)] = pltpu.async_copy(
                rows[b][par],
                out_hbm.at[pl.ds(b * SEQ + pos_base + h * CHUNK, CHUNK)],
                osem[b][par])

    gathers(0)
    gathers(1)
    pload.wait()
    for h in range(NGRP):
        par = h % 3
        if h + 2 < NGRP:
            if h >= 1:
                for b in range(BATCH):
                    o[(b, h - 1)].wait()
            gathers(h + 2)
        for b in range(BATCH):
            g[(b, h)].wait()

        def add_group(par=par, h=h):
            @plsc.parallel_loop(0, CHUNK)
            def _row(r):
                rbufs = [rows[b][par].at[r] for b in range(BATCH)]
                pr = pos_v.at[h * CHUNK + r]
                @plsc.parallel_loop(0, DIM, step=LANES, unroll=4)
                def _slice(dd):
                    sl = pl.ds(dd, LANES)
                    pv = pr[sl]
                    for b in range(BATCH):
                        rbufs[b][sl] = rbufs[b][sl] + pv

        add_group()
        outs(h)
    for b in range(BATCH):
        o[(b, NGRP - 3)].wait()
        o[(b, NGRP - 2)].wait()
        o[(b, NGRP - 1)].wait()


@jax.jit
def kernel(input_ids, vocab_W, pos_W):
    ids = input_ids.astype(jnp.int32)
    mesh = plsc.VectorSubcoreMesh(core_axis_name="c", subcore_axis_name="s")
    run = pl.kernel(
        _body,
        out_type=jax.ShapeDtypeStruct((BATCH * SEQ, DIM), jnp.float32),
        mesh=mesh,
        scratch_types=(
            [pltpu.VMEM((BATCH, POSB), jnp.int32),
             pltpu.VMEM((POSB, DIM), jnp.float32)]
            + [pltpu.VMEM((CHUNK, DIM), jnp.float32)
               for _ in range(3 * BATCH)]
            + [pltpu.SemaphoreType.DMA for _ in range(2 + 6 * BATCH)]
        ),
    )
    out = run(ids, vocab_W, pos_W)
    return out.reshape(BATCH, SEQ, DIM)


# speedup vs baseline: 1.2417x; 1.0020x over previous
import jax
import jax.numpy as jnp
from jax import lax
from jax.experimental import pallas as pl
from jax.experimental.pallas import tpu as pltpu
from jax.experimental.pallas import tpu_sc as plsc

VOCAB = 100000
DIM = 768
SEQ = 2048
BATCH = 4

NC = 2
NS = 16
LANES = 16
NW = NC * NS
POSB = SEQ // NW
CHUNK = 8
NGRP = POSB // CHUNK
DSLICES = DIM // LANES


def _body(ids_hbm, vocab_hbm, pos_hbm, out_hbm, idx_v, pos_v,
          r00, r01, r02, r10, r11, r12, r20, r21, r22, r30, r31, r32,
          psem, isem,
          gs00, gs01, gs02, gs10, gs11, gs12, gs20, gs21, gs22,
          gs30, gs31, gs32,
          os00, os01, os02, os10, os11, os12, os20, os21, os22,
          os30, os31, os32):
    rows = [[r00, r01, r02], [r10, r11, r12], [r20, r21, r22],
            [r30, r31, r32]]
    gsem = [[gs00, gs01, gs02], [gs10, gs11, gs12], [gs20, gs21, gs22],
            [gs30, gs31, gs32]]
    osem = [[os00, os01, os02], [os10, os11, os12], [os20, os21, os22],
            [os30, os31, os32]]

    c_i = lax.axis_index("c")
    s_i = lax.axis_index("s")
    wid = s_i * NC + c_i
    pos_base = wid * POSB

    pload = pltpu.async_copy(pos_hbm.at[pl.ds(pos_base, POSB)], pos_v, psem)
    iloads = [pltpu.async_copy(ids_hbm.at[b, pl.ds(pos_base, POSB)],
                               idx_v.at[b], isem) for b in range(BATCH)]
    for il in iloads:
        il.wait()

    g, o = {}, {}

    def gathers(h):
        par = h % 3
        for b in range(BATCH):
            g[(b, h)] = pltpu.async_copy(
                vocab_hbm.at[idx_v.at[b, pl.ds(h * CHUNK, CHUNK)]],
                rows[b][par], gsem[b][par])

    def outs(h):
        par = h % 3
        for b in range(BATCH):
            o[(b, h)] = pltpu.async_copy(
                rows[b][par],
                out_hbm.at[pl.ds(b * SEQ + pos_base + h * CHUNK, CHUNK)],
                osem[b][par])

    gathers(0)
    gathers(1)
    pload.wait()
    for h in range(NGRP):
        par = h % 3
        if h + 2 < NGRP:
            if h >= 1:
                for b in range(BATCH):
                    o[(b, h - 1)].wait()
            gathers(h + 2)
        for b in range(BATCH):
            g[(b, h)].wait()

        def add_group(par=par, h=h):
            @plsc.parallel_loop(0, CHUNK)
            def _row(r):
                rbufs = [rows[b][par].at[r] for b in range(BATCH)]
                pr = pos_v.at[h * CHUNK + r]
                @plsc.parallel_loop(0, DIM, step=LANES, unroll=8)
                def _slice(dd):
                    sl = pl.ds(dd, LANES)
                    pv = pr[sl]
                    for b in range(BATCH):
                        rbufs[b][sl] = rbufs[b][sl] + pv

        add_group()
        outs(h)
    for b in range(BATCH):
        o[(b, NGRP - 3)].wait()
        o[(b, NGRP - 2)].wait()
        o[(b, NGRP - 1)].wait()


@jax.jit
def kernel(input_ids, vocab_W, pos_W):
    ids = input_ids.astype(jnp.int32)
    mesh = plsc.VectorSubcoreMesh(core_axis_name="c", subcore_axis_name="s")
    run = pl.kernel(
        _body,
        out_type=jax.ShapeDtypeStruct((BATCH * SEQ, DIM), jnp.float32),
        mesh=mesh,
        scratch_types=(
            [pltpu.VMEM((BATCH, POSB), jnp.int32),
             pltpu.VMEM((POSB, DIM), jnp.float32)]
            + [pltpu.VMEM((CHUNK, DIM), jnp.float32)
               for _ in range(3 * BATCH)]
            + [pltpu.SemaphoreType.DMA for _ in range(2 + 6 * BATCH)]
        ),
    )
    out = run(ids, vocab_W, pos_W)
    return out.reshape(BATCH, SEQ, DIM)
